# baseline (speedup 1.0000x reference)
import jax
import jax.numpy as jnp
from jax.experimental import pallas as pl

NUM_CLASSES = 5
BLOCK = 2048


def _body(seq_ref, out_ref):
    s = seq_ref[:, :]
    classes = jax.lax.broadcasted_iota(jnp.int32, (NUM_CLASSES, 8, BLOCK), 0)
    out_ref[:, :, :] = (s[None, :, :] == classes).astype(jnp.float32)


def kernel(seq):
    L = seq.shape[0]
    W = L // 8
    seq2 = seq.reshape(8, W)
    grid = pl.cdiv(W, BLOCK)
    out = pl.pallas_call(
        _body,
        grid=(grid,),
        in_specs=[pl.BlockSpec((8, BLOCK), lambda i: (0, i))],
        out_specs=pl.BlockSpec((NUM_CLASSES, 8, BLOCK), lambda i: (0, 0, i)),
        out_shape=jax.ShapeDtypeStruct((NUM_CLASSES, 8, W), jnp.float32),
    )(seq2)
    return out.reshape(NUM_CLASSES, L)


# speedup vs baseline: 1.3646x; 1.3646x over previous
import jax
import jax.numpy as jnp
from jax import lax
from jax.experimental import pallas as pl
from jax.experimental.pallas import tpu as pltpu
from jax.experimental.pallas import tpu_sc as plsc

_C = 5
_S = 8192
_LANES = 16


def _sc_body(L, NB, T, NW, seq_hbm, out_hbm, seq_v, out_v):
    wid = lax.axis_index("s") * 2 + lax.axis_index("c")
    last_base = L - _S
    for t in range(T):
        b = wid + NW * t

        @pl.when(b < NB)
        def _():
            base = jnp.minimum(b * _S, last_base)
            base = pl.multiple_of(base, 8)
            pltpu.sync_copy(seq_hbm.at[pl.ds(base, _S)], seq_v)

            def j_body(j, carry):
                s = seq_v[pl.ds(j * _LANES, _LANES)]
                for c in range(_C):
                    out_v[pl.ds(c * _S + j * _LANES, _LANES)] = jnp.where(
                        s == c, 1.0, 0.0
                    ).astype(jnp.float32)
                return carry

            lax.fori_loop(0, _S // _LANES, j_body, 0)
            for c in range(_C):
                pltpu.sync_copy(
                    out_v.at[pl.ds(c * _S, _S)],
                    out_hbm.at[pl.ds(c * L + base, _S)],
                )


def kernel(seq):
    import functools

    L = seq.shape[0]
    NW = 32
    NB = pl.cdiv(L, _S)
    T = pl.cdiv(NB, NW)
    mesh = plsc.VectorSubcoreMesh(core_axis_name="c", subcore_axis_name="s")
    body = functools.partial(_sc_body, L, NB, T, NW)
    flat = pl.kernel(
        body,
        mesh=mesh,
        out_type=jax.ShapeDtypeStruct((_C * L,), jnp.float32),
        scratch_types=[
            pltpu.VMEM((_S,), jnp.int32),
            pltpu.VMEM((_C * _S,), jnp.float32),
        ],
    )(seq)
    return flat.reshape(_C, L)


# speedup vs baseline: 14.2996x; 10.4790x over previous
import functools

import jax
import jax.numpy as jnp
from jax import lax
from jax.experimental import pallas as pl
from jax.experimental.pallas import tpu as pltpu
from jax.experimental.pallas import tpu_sc as plsc

_C = 5
_S = 8192
_LANES = 16


def _do_chunk(seq_hbm, out_hbm, seq_v, out_v, sem, base, n):
    pltpu.sync_copy(seq_hbm.at[pl.ds(base, n)], seq_v.at[pl.ds(0, n)])

    def j_body(j, carry):
        s = seq_v[pl.ds(j * _LANES, _LANES)]
        for c in range(_C):
            out_v[c, pl.ds(j * _LANES, _LANES)] = jnp.where(
                s == c, 1.0, 0.0
            ).astype(jnp.float32)
        return carry

    lax.fori_loop(0, n // _LANES, j_body, 0)
    copies = [
        pltpu.async_copy(
            out_v.at[:, pl.ds(k, 128)],
            out_hbm.at[:, pl.ds(base + k, 128)],
            sem,
        )
        for k in range(0, n, 128)
    ]
    for cp in copies:
        cp.wait()


def _sc_body(L, NB, T, NW, seq_hbm, out_hbm, seq_v, out_v, sem):
    wid = lax.axis_index("s") * 2 + lax.axis_index("c")
    for t in range(T):
        b = wid + NW * t

        @pl.when(b < NB)
        def _():
            base = pl.multiple_of(b * _S, 128)
            _do_chunk(seq_hbm, out_hbm, seq_v, out_v, sem, base, _S)

    tail0 = NB * _S
    rem = (L // 128) * 128 - tail0
    if rem:

        @pl.when(wid == NW - 1)
        def _():
            _do_chunk(seq_hbm, out_hbm, seq_v, out_v, sem, tail0, rem)


def _tail_body(seq_ref, _sc_ref, out_ref):
    s = seq_ref[:]
    classes = jax.lax.broadcasted_iota(jnp.int32, (_C, 128), 0)
    out_ref[:, :] = (s[None, :] == classes).astype(jnp.float32)


def kernel(seq):
    L = seq.shape[0]
    NW = 32
    NB = ((L // 128) * 128) // _S
    T = pl.cdiv(NB, NW)
    mesh = plsc.VectorSubcoreMesh(core_axis_name="c", subcore_axis_name="s")
    body = functools.partial(_sc_body, L, NB, T, NW)
    sc_out = pl.kernel(
        body,
        mesh=mesh,
        out_type=jax.ShapeDtypeStruct((_C, L), jnp.float32),
        scratch_types=[
            pltpu.VMEM((_S,), jnp.int32),
            pltpu.VMEM((_C, _S), jnp.float32),
            pltpu.SemaphoreType.DMA,
        ],
    )(seq)
    if L % 128 == 0:
        return sc_out
    j = L // 128
    return pl.pallas_call(
        _tail_body,
        grid=(1,),
        in_specs=[
            pl.BlockSpec((128,), lambda i: (j,)),
            pl.BlockSpec(memory_space=pl.ANY),
        ],
        out_specs=pl.BlockSpec((_C, 128), lambda i: (0, j)),
        out_shape=jax.ShapeDtypeStruct((_C, L), jnp.float32),
        input_output_aliases={1: 0},
    )(seq, sc_out)


# speedup vs baseline: 17.2335x; 1.2052x over previous
import functools

import jax
import jax.numpy as jnp
from jax import lax
from jax.experimental import pallas as pl
from jax.experimental.pallas import tpu as pltpu
from jax.experimental.pallas import tpu_sc as plsc

_C = 5
_S = 4096
_Q = 8
_LANES = 16


def _compute_block(seq_v, out_v, q, pb):
    def j_body(j, carry):
        s = seq_v[pl.ds(q * _S + j * _LANES, _LANES)]
        for c in range(_C):
            out_v[pb, c, pl.ds(j * _LANES, _LANES)] = jnp.where(
                s == c, 1.0, 0.0
            ).astype(jnp.float32)
        return carry

    lax.fori_loop(0, _S // _LANES, j_body, 0, unroll=4)


def _sc_body(L, NW, stride, seq_hbm, out_hbm, seq_v, out_v, sem0, sem1):
    wid = lax.axis_index("s") * 2 + lax.axis_index("c")
    span = _Q * _S
    aligned = (L // 128) * 128
    base = pl.multiple_of(jnp.minimum(wid * stride, aligned - span), 128)
    sem = (sem0, sem1)

    pltpu.sync_copy(seq_hbm.at[pl.ds(base, span)], seq_v)

    out_cps = [[], []]
    for q in range(_Q):
        pb = q % 2
        for cp in out_cps[pb]:
            cp.wait()
        _compute_block(seq_v, out_v, q, pb)
        col0 = pl.multiple_of(base + q * _S, 128)
        out_cps[pb] = [
            pltpu.async_copy(
                out_v.at[pb, :, pl.ds(k, 128)],
                out_hbm.at[:, pl.ds(col0 + k, 128)],
                sem[pb],
            )
            for k in range(0, _S, 128)
        ]
    for pb in range(2):
        for cp in out_cps[pb]:
            cp.wait()


def _tail_body(seq_ref, _sc_ref, out_ref):
    s = seq_ref[:]
    classes = jax.lax.broadcasted_iota(jnp.int32, (_C, 128), 0)
    out_ref[:, :] = (s[None, :] == classes).astype(jnp.float32)


def kernel(seq):
    L = seq.shape[0]
    NW = 32
    span = _Q * _S
    aligned = (L // 128) * 128
    stride = -(-(aligned - span) // (NW - 1))
    stride = -(-stride // 128) * 128
    assert stride <= span
    mesh = plsc.VectorSubcoreMesh(core_axis_name="c", subcore_axis_name="s")
    body = functools.partial(_sc_body, L, NW, stride)
    sc_out = pl.kernel(
        body,
        mesh=mesh,
        out_type=jax.ShapeDtypeStruct((_C, L), jnp.float32),
        scratch_types=[
            pltpu.VMEM((span,), jnp.int32),
            pltpu.VMEM((2, _C, _S), jnp.float32),
            pltpu.SemaphoreType.DMA,
            pltpu.SemaphoreType.DMA,
        ],
    )(seq)
    if L % 128 == 0:
        return sc_out
    j = L // 128
    return pl.pallas_call(
        _tail_body,
        grid=(1,),
        in_specs=[
            pl.BlockSpec((128,), lambda i: (j,)),
            pl.BlockSpec(memory_space=pl.ANY),
        ],
        out_specs=pl.BlockSpec((_C, 128), lambda i: (0, j)),
        out_shape=jax.ShapeDtypeStruct((_C, L), jnp.float32),
        input_output_aliases={1: 0},
    )(seq, sc_out)
